# Initial kernel scaffold; baseline (speedup 1.0000x reference)
import jax
import jax.numpy as jnp
from jax.experimental import pallas as pl


def kernel(ell, theta, field_strengths, masses, frozen):
    raise NotImplementedError("write your pallas kernel here")


# speedup vs baseline: 1.7057x; 1.7057x over previous
import jax
import jax.numpy as jnp
import numpy as np
from jax.experimental import pallas as pl

_PHI = float((1.0 + np.sqrt(5.0)) / 2.0)
_EPS = 1e-10
_GRID = 64
_ELL_MAX = 10.0
_TAU = float(2.0 * np.pi)
_PI = float(np.pi)

_BI = 256


def _body(ell_l, th_l, fs_l, act_l, ell_c, th_c, fs_c, m_c, act_c,
          fell_o, fth_o, cnt_o):
    e = ell_l[:]
    t = th_l[:]
    fsj = fs_l[:] * act_l[:]
    ei = ell_c[:]
    ti = th_c[:]
    fsi = fs_c[:] * act_c[:]
    mi = m_c[:]

    d_ell = e[None, :] - ei
    x = t[None, :] - ti + _PI
    d_th = jnp.remainder(x, _TAU) - _PI

    a = jnp.abs(d_ell) + 1e-12
    b = jnp.abs(d_th) + 1e-12
    la = jnp.log2(a)
    lb = jnp.log2(b)
    s = jnp.exp2(la * _PHI) + jnp.exp2(lb * _PHI)
    ls = jnp.log2(s)
    p = jnp.exp2(ls * _PHI)
    inv_dl = jnp.exp2(ls * (-1.0 / _PHI))
    w = (fsi * fsj[None, :]) / (p * mi + _EPS) * inv_dl
    fell_o[:] = jnp.sum(w * d_ell, axis=1, keepdims=True)
    fth_o[:] = jnp.sum(w * d_th, axis=1, keepdims=True)

    i = pl.program_id(0)
    ce = jnp.clip(
        jnp.floor(jnp.clip(e, 0.0, _ELL_MAX) / _ELL_MAX * _GRID).astype(jnp.int32),
        0, _GRID - 1)
    ct = jnp.clip(
        jnp.floor(jnp.mod(t, _TAU) / _TAU * _GRID).astype(jnp.int32),
        0, _GRID - 1)
    h = ce * _GRID + ct
    bins = i * _BI + jax.lax.broadcasted_iota(jnp.int32, (_BI, 1), 0)
    cnt_o[:] = jnp.sum((h[None, :] == bins).astype(jnp.int32),
                       axis=1, keepdims=True)


def kernel(ell, theta, field_strengths, masses, frozen):
    n = ell.shape[0]
    act = (~frozen).astype(jnp.float32)
    col = lambda v: v.reshape(n, 1)
    full = pl.BlockSpec((n,), lambda i: (0,))
    blk = pl.BlockSpec((_BI, 1), lambda i: (i, 0))
    fell, fth, cnt = pl.pallas_call(
        _body,
        grid=(n // _BI,),
        in_specs=[full, full, full, full, blk, blk, blk, blk, blk],
        out_specs=(blk, blk, blk),
        out_shape=(
            jax.ShapeDtypeStruct((n, 1), jnp.float32),
            jax.ShapeDtypeStruct((n, 1), jnp.float32),
            jax.ShapeDtypeStruct((n, 1), jnp.int32),
        ),
    )(ell, theta, field_strengths, act,
      col(ell), col(theta), col(field_strengths), col(masses), col(act))
    forces = jnp.stack([fell[:, 0], fth[:, 0]], axis=0)
    return forces, cnt[:, 0]


# speedup vs baseline: 1.9727x; 1.1565x over previous
import jax
import jax.numpy as jnp
import numpy as np
from jax.experimental import pallas as pl

_PHI = float((1.0 + np.sqrt(5.0)) / 2.0)
_EPS = 1e-10
_GRID = 64
_ELL_MAX = 10.0
_TAU = float(2.0 * np.pi)
_PI = float(np.pi)

_BI = 256


def _body(ell_l, th_l, fs_l, act_l, ell_c, th_c, fs_c, m_c, act_c,
          fell_o, fth_o, cnt_o):
    e = ell_l[:]
    t = th_l[:]
    fsj = fs_l[:] * act_l[:]
    ei = ell_c[:]
    ti = th_c[:]
    fsi = fs_c[:] * act_c[:]
    mi = m_c[:]

    d_ell = e[None, :] - ei
    y = t[None, :] - ti + _PI
    r = y + (jnp.where(y < 0.0, _TAU, 0.0) - jnp.where(y >= _TAU, _TAU, 0.0))
    d_th = r - _PI

    a = jnp.abs(d_ell) + 1e-12
    b = jnp.abs(d_th) + 1e-12
    la = jnp.log2(a)
    lb = jnp.log2(b)
    s = jnp.exp2(la * _PHI) + jnp.exp2(lb * _PHI)
    q = jnp.exp2(jnp.log2(s) * (1.0 / _PHI))
    den = (s * q) * (q * mi) + _EPS * q
    w = (fsi * fsj[None, :]) / den
    fell_o[:] = jnp.sum(w * d_ell, axis=1, keepdims=True)
    fth_o[:] = jnp.sum(w * d_th, axis=1, keepdims=True)

    i = pl.program_id(0)
    ce = jnp.clip(
        jnp.floor(jnp.clip(e, 0.0, _ELL_MAX) / _ELL_MAX * _GRID).astype(jnp.int32),
        0, _GRID - 1)
    ct = jnp.clip(
        jnp.floor(jnp.mod(t, _TAU) / _TAU * _GRID).astype(jnp.int32),
        0, _GRID - 1)
    h = ce * _GRID + ct
    bins = i * _BI + jax.lax.broadcasted_iota(jnp.int32, (_BI, 1), 0)
    cnt_o[:] = jnp.sum((h[None, :] == bins).astype(jnp.int32),
                       axis=1, keepdims=True)


def kernel(ell, theta, field_strengths, masses, frozen):
    n = ell.shape[0]
    act = (~frozen).astype(jnp.float32)
    col = lambda v: v.reshape(n, 1)
    full = pl.BlockSpec((n,), lambda i: (0,))
    blk = pl.BlockSpec((_BI, 1), lambda i: (i, 0))
    fell, fth, cnt = pl.pallas_call(
        _body,
        grid=(n // _BI,),
        in_specs=[full, full, full, full, blk, blk, blk, blk, blk],
        out_specs=(blk, blk, blk),
        out_shape=(
            jax.ShapeDtypeStruct((n, 1), jnp.float32),
            jax.ShapeDtypeStruct((n, 1), jnp.float32),
            jax.ShapeDtypeStruct((n, 1), jnp.int32),
        ),
    )(ell, theta, field_strengths, act,
      col(ell), col(theta), col(field_strengths), col(masses), col(act))
    forces = jnp.stack([fell[:, 0], fth[:, 0]], axis=0)
    return forces, cnt[:, 0]


# speedup vs baseline: 1.9956x; 1.0116x over previous
import jax
import jax.numpy as jnp
import numpy as np
from jax.experimental import pallas as pl

_PHI = float((1.0 + np.sqrt(5.0)) / 2.0)
_EPS = 1e-10
_GRID = 64
_ELL_MAX = 10.0
_TAU = float(2.0 * np.pi)
_PI = float(np.pi)

_BI = 512


def _body(ell_l, th_l, fs_l, act_l, ell_c, th_c, fs_c, m_c, act_c,
          fell_o, fth_o, cnt_o):
    e = ell_l[:]
    t = th_l[:]
    fsj = fs_l[:] * act_l[:]
    ei = ell_c[:]
    ti = th_c[:]
    fsi = fs_c[:] * act_c[:]
    mi = m_c[:]

    d_ell = e[None, :] - ei
    y = t[None, :] - ti + _PI
    r = y + (jnp.where(y < 0.0, _TAU, 0.0) - jnp.where(y >= _TAU, _TAU, 0.0))
    d_th = r - _PI

    a = jnp.abs(d_ell) + 1e-12
    b = jnp.abs(d_th) + 1e-12
    la = jnp.log2(a)
    lb = jnp.log2(b)
    s = jnp.exp2(la * _PHI) + jnp.exp2(lb * _PHI)
    q = jnp.exp2(jnp.log2(s) * (1.0 / _PHI))
    den = (s * q) * (q * mi) + _EPS * q
    w = (fsi * fsj[None, :]) / den
    fell_o[:] = jnp.sum(w * d_ell, axis=1, keepdims=True)
    fth_o[:] = jnp.sum(w * d_th, axis=1, keepdims=True)

    i = pl.program_id(0)
    ce = jnp.clip(
        jnp.floor(jnp.clip(e, 0.0, _ELL_MAX) / _ELL_MAX * _GRID).astype(jnp.int32),
        0, _GRID - 1)
    ct = jnp.clip(
        jnp.floor(jnp.mod(t, _TAU) / _TAU * _GRID).astype(jnp.int32),
        0, _GRID - 1)
    h = ce * _GRID + ct
    bins = i * _BI + jax.lax.broadcasted_iota(jnp.int32, (_BI, 1), 0)
    cnt_o[:] = jnp.sum((h[None, :] == bins).astype(jnp.int32),
                       axis=1, keepdims=True)


def kernel(ell, theta, field_strengths, masses, frozen):
    n = ell.shape[0]
    act = (~frozen).astype(jnp.float32)
    col = lambda v: v.reshape(n, 1)
    full = pl.BlockSpec((n,), lambda i: (0,))
    blk = pl.BlockSpec((_BI, 1), lambda i: (i, 0))
    fell, fth, cnt = pl.pallas_call(
        _body,
        grid=(n // _BI,),
        in_specs=[full, full, full, full, blk, blk, blk, blk, blk],
        out_specs=(blk, blk, blk),
        out_shape=(
            jax.ShapeDtypeStruct((n, 1), jnp.float32),
            jax.ShapeDtypeStruct((n, 1), jnp.float32),
            jax.ShapeDtypeStruct((n, 1), jnp.int32),
        ),
    )(ell, theta, field_strengths, act,
      col(ell), col(theta), col(field_strengths), col(masses), col(act))
    forces = jnp.stack([fell[:, 0], fth[:, 0]], axis=0)
    return forces, cnt[:, 0]
